# baseline (speedup 1.0000x reference)
import functools

import jax
import jax.numpy as jnp
from jax import lax
from jax.experimental import pallas as pl
from jax.experimental.pallas import tpu as pltpu
from jax.experimental.pallas import tpu_sc as plsc


def kernel(x, symbol_library):
    batch, seq_len, d_model = x.shape
    num_workers = 32
    rows_per_worker = seq_len // num_workers
    chunks = (48, 48, 16, 16)
    assert sum(chunks) == rows_per_worker
    starts = [sum(chunks[:i]) for i in range(len(chunks))]
    n_chunks = len(chunks)
    nbuf = 2
    bufrows = max(chunks)

    mesh = plsc.VectorSubcoreMesh(core_axis_name="c", subcore_axis_name="s")

    @functools.partial(
        pl.kernel,
        mesh=mesh,
        out_type=jax.ShapeDtypeStruct((batch, seq_len, d_model), x.dtype),
        scratch_types=[
            pltpu.VMEM((nbuf, bufrows, d_model), jnp.float32),
            pltpu.SemaphoreType.DMA,
            pltpu.SemaphoreType.DMA,
        ],
    )
    def body(table_hbm, out_hbm, bufs, rsem, wsem):
        wid = lax.axis_index("c") * 16 + lax.axis_index("s")
        base = wid * rows_per_worker
        border = lax.axis_index("s")

        def start_read(c):
            return pltpu.async_copy(
                table_hbm.at[pl.ds(base + starts[c], chunks[c])],
                bufs.at[c % nbuf, pl.ds(0, chunks[c])], rsem)

        reads = {0: start_read(0)}
        writes = {}
        for c in range(n_chunks):
            reads[c].wait()
            if c + 1 < n_chunks:
                if c + 1 >= nbuf:
                    for w in writes.pop(c + 1 - nbuf):
                        w.wait()
                reads[c + 1] = start_read(c + 1)
            writes[c] = [
                pltpu.async_copy(
                    bufs.at[c % nbuf, pl.ds(0, chunks[c])],
                    out_hbm.at[lax.rem(border + j, batch),
                               pl.ds(base + starts[c], chunks[c])], wsem)
                for j in range(batch)
            ]
        for c in sorted(writes):
            for w in writes[c]:
                w.wait()

    return body(symbol_library)
